# Initial kernel scaffold; baseline (speedup 1.0000x reference)
import jax
import jax.numpy as jnp
from jax.experimental import pallas as pl


def kernel(node_features, edge_index, edge_features, Wn, bn, We, be, Wi, Wj, Weij, attn_W0, attn_b0, attn_W1, attn_b1, attn_W2, attn_b2, attn_param, node_W0, node_b0, node_W1, node_b1, node_W2, node_b2, edge_W0, edge_b0, edge_W1, edge_b1, edge_W2, edge_b2, lin_W, Wg, bg):
    raise NotImplementedError("write your pallas kernel here")


# speedup vs baseline: 15.2694x; 15.2694x over previous
import functools

import jax
import jax.numpy as jnp
from jax import lax
from jax.experimental import pallas as pl
from jax.experimental.pallas import tpu as pltpu
from jax.experimental.pallas import tpu_sc as plsc

N = 10000
E = 320000
DN = 128
DE = 16
C = 32
H = 3
HC = H * C
W = 128

NC, NS = 2, 16
GW = 128
IR = E // GW
ACC = 10240
TRASH = N

PREC = lax.Precision.HIGHEST
F32 = jnp.float32


def _mesh():
    return plsc.VectorSubcoreMesh(
        core_axis_name="c", subcore_axis_name="s", num_cores=NC, num_subcores=NS
    )


def _dot(a, b):
    return jnp.dot(a, b, preferred_element_type=F32, precision=PREC)


def _pad128(x, bn):
    return jnp.concatenate([x, jnp.zeros((bn, W - x.shape[1]), F32)], axis=1)


def _sc_gather2(td, tj, dst2, src2):
    @functools.partial(
        pl.kernel,
        out_type=(jax.ShapeDtypeStruct((E, W), F32),) * 2,
        mesh=_mesh(),
    )
    def k(td_h, tj_h, d_h, s_h, gd_h, xj_h):
        def body(d_v, s_v, gd_v, xj_v):
            pltpu.sync_copy(td_h.at[d_v.at[0]], gd_v)
            pltpu.sync_copy(tj_h.at[s_v.at[0]], xj_v)

        pltpu.emit_pipeline(
            body,
            grid=(IR,),
            in_specs=[
                pl.BlockSpec((1, GW), lambda i: (i, 0)),
                pl.BlockSpec((1, GW), lambda i: (i, 0)),
            ],
            out_specs=[pl.BlockSpec((GW, W), lambda i: (i, 0))] * 2,
            core_axis_name=("c", "s"),
            dimension_semantics=(pltpu.PARALLEL,),
        )(d_h, s_h, gd_h, xj_h)

    return k(td, tj, dst2, src2)


def _sc_gather1(tab, idx2):
    @functools.partial(
        pl.kernel,
        out_type=jax.ShapeDtypeStruct((E, W), F32),
        mesh=_mesh(),
    )
    def k(t_h, i_h, o_h):
        def body(i_v, o_v):
            pltpu.sync_copy(t_h.at[i_v.at[0]], o_v)

        pltpu.emit_pipeline(
            body,
            grid=(IR,),
            in_specs=[pl.BlockSpec((1, GW), lambda i: (i, 0))],
            out_specs=[pl.BlockSpec((GW, W), lambda i: (i, 0))],
            core_axis_name=("c", "s"),
            dimension_semantics=(pltpu.PARALLEL,),
        )(i_h, o_h)

    return k(tab, idx2)


def _sc_scatter_rows(msg, dst2, zrows):
    @functools.partial(
        pl.kernel,
        out_type=jax.ShapeDtypeStruct((NC, ACC, W), F32),
        mesh=_mesh(),
        scratch_types=[pltpu.VMEM_SHARED((ACC, W), F32)],
    )
    def k(msg_h, d_h, z_h, acc_o, acc_s):
        cid = lax.axis_index("c")
        sid = lax.axis_index("s")
        r0 = sid * (ACC // NS)
        pltpu.sync_copy(z_h, acc_s.at[pl.ds(r0, ACC // NS)])
        plsc.subcore_barrier()

        def body(msg_v, d_v):
            pltpu.sync_copy(msg_v, acc_s.at[d_v.at[0]], add=True)

        pltpu.emit_pipeline(
            body,
            grid=(IR,),
            in_specs=[
                pl.BlockSpec((GW, W), lambda i: (i, 0)),
                pl.BlockSpec((1, GW), lambda i: (i, 0)),
            ],
            core_axis_name=("c", "s"),
            dimension_semantics=(pltpu.PARALLEL,),
        )(msg_h, d_h)
        plsc.subcore_barrier()
        pltpu.sync_copy(acc_s.at[pl.ds(r0, ACC // NS)],
                        acc_o.at[cid, pl.ds(r0, ACC // NS)])

    return k(msg, dst2, zrows)


def _sc_degree(srcp2, zrows, ones_h):
    @functools.partial(
        pl.kernel,
        out_type=jax.ShapeDtypeStruct((NC, ACC, W), F32),
        mesh=_mesh(),
        scratch_types=[
            pltpu.VMEM_SHARED((ACC, W), F32),
            pltpu.VMEM((GW, W), F32),
        ],
    )
    def k(sp_h, z_h, o_h, deg_o, deg_s, ones_v):
        cid = lax.axis_index("c")
        sid = lax.axis_index("s")
        r0 = sid * (ACC // NS)
        pltpu.sync_copy(z_h, deg_s.at[pl.ds(r0, ACC // NS)])
        pltpu.sync_copy(o_h, ones_v)
        plsc.subcore_barrier()

        def body(sp_v):
            pltpu.sync_copy(ones_v, deg_s.at[sp_v.at[0]], add=True)

        pltpu.emit_pipeline(
            body,
            grid=(IR,),
            in_specs=[pl.BlockSpec((1, GW), lambda i: (i, 0))],
            core_axis_name=("c", "s"),
            dimension_semantics=(pltpu.PARALLEL,),
        )(sp_h)
        plsc.subcore_barrier()
        pltpu.sync_copy(deg_s.at[pl.ds(r0, ACC // NS)],
                        deg_o.at[cid, pl.ds(r0, ACC // NS)])

    return k(srcp2, zrows, ones_h)


def _sc_spmm(y, src2, dstp2, zrows):
    @functools.partial(
        pl.kernel,
        out_type=jax.ShapeDtypeStruct((NC, ACC, W), F32),
        mesh=_mesh(),
        scratch_types=[
            pltpu.VMEM_SHARED((ACC, W), F32),
            pltpu.VMEM((GW, W), F32),
        ],
    )
    def k(y_h, s_h, d_h, z_h, z_o, z_s, buf_v):
        cid = lax.axis_index("c")
        sid = lax.axis_index("s")
        r0 = sid * (ACC // NS)
        pltpu.sync_copy(z_h, z_s.at[pl.ds(r0, ACC // NS)])
        plsc.subcore_barrier()

        def body(s_v, d_v):
            pltpu.sync_copy(y_h.at[s_v.at[0]], buf_v)
            pltpu.sync_copy(buf_v, z_s.at[d_v.at[0]], add=True)

        pltpu.emit_pipeline(
            body,
            grid=(IR,),
            in_specs=[
                pl.BlockSpec((1, GW), lambda i: (i, 0)),
                pl.BlockSpec((1, GW), lambda i: (i, 0)),
            ],
            core_axis_name=("c", "s"),
            dimension_semantics=(pltpu.PARALLEL,),
        )(s_h, d_h)
        plsc.subcore_barrier()
        pltpu.sync_copy(z_s.at[pl.ds(r0, ACC // NS)],
                        z_o.at[cid, pl.ds(r0, ACC // NS)])

    return k(y, src2, dstp2, zrows)


def _tc_node_pre(nf, Wj, A00, Wi, Wn, bn1):
    BN = 2000

    def body(nf_r, wj_r, a0_r, wi_r, wn_r, bn_r, td_r, tj_r, nl_r):
        x = nf_r[...]
        td_r[...] = _pad128(_dot(_dot(x, wj_r[...]), a0_r[...]), BN)
        tj_r[...] = _pad128(_dot(x, wi_r[...]), BN)
        nl_r[...] = _dot(x, wn_r[...]) + bn_r[...]

    full = lambda s: pl.BlockSpec(s, lambda i: (0,) * len(s))
    return pl.pallas_call(
        body,
        grid=(N // BN,),
        in_specs=[
            pl.BlockSpec((BN, DN), lambda i: (i, 0)),
            full((DN, HC)), full((HC, HC)), full((DN, HC)),
            full((DN, C)), full((1, C)),
        ],
        out_specs=[
            pl.BlockSpec((BN, W), lambda i: (i, 0)),
            pl.BlockSpec((BN, W), lambda i: (i, 0)),
            pl.BlockSpec((BN, C), lambda i: (i, 0)),
        ],
        out_shape=[
            jax.ShapeDtypeStruct((N, W), F32),
            jax.ShapeDtypeStruct((N, W), F32),
            jax.ShapeDtypeStruct((N, C), F32),
        ],
    )(nf, Wj, A00, Wi, Wn, bn1)


def _tc_edge_pre(ef, Weij, A01, We, b01, be1):
    BE = 6400

    def body(ef_r, weij_r, a1_r, we_r, b0_r, be_r, ge_r, el_r):
        x = ef_r[...]
        ge_r[...] = _dot(_dot(x, weij_r[...]), a1_r[...]) + b0_r[...]
        el_r[...] = _dot(x, we_r[...]) + be_r[...]

    full = lambda s: pl.BlockSpec(s, lambda i: (0,) * len(s))
    return pl.pallas_call(
        body,
        grid=(E // BE,),
        in_specs=[
            pl.BlockSpec((BE, DE), lambda i: (i, 0)),
            full((DE, HC)), full((HC, HC)), full((DE, C)),
            full((1, HC)), full((1, C)),
        ],
        out_specs=[
            pl.BlockSpec((BE, HC), lambda i: (i, 0)),
            pl.BlockSpec((BE, C), lambda i: (i, 0)),
        ],
        out_shape=[
            jax.ShapeDtypeStruct((E, HC), F32),
            jax.ShapeDtypeStruct((E, C), F32),
        ],
    )(ef, Weij, A01, We, b01, be1)


def _tc_redirect(src2, dst2):
    def body(s_r, d_r, sp_r, dp_r):
        s = s_r[...]
        d = d_r[...]
        loop = s == d
        sp_r[...] = jnp.where(loop, TRASH, s)
        dp_r[...] = jnp.where(loop, TRASH, d)

    whole = pl.BlockSpec((IR, GW), lambda: (0, 0))
    return pl.pallas_call(
        body,
        grid=(),
        in_specs=[whole, whole],
        out_specs=[whole, whole],
        out_shape=[jax.ShapeDtypeStruct((IR, GW), jnp.int32)] * 2,
    )(src2, dst2)


def _tc_edge_mlp(gd, xj, ge, el, A02p, W1, b11, W2, b21, ap1,
                 eW0, eb01, eW1, eb11, eW2, eb21):
    BE = 1600

    def body(gd_r, xj_r, ge_r, el_r, a2_r, w1_r, b1_r, w2_r, b2_r, ap_r,
             ew0_r, eb0_r, ew1_r, eb1_r, ew2_r, eb2_r,
             msg_r, ex_r, eo_r):
        xj = xj_r[...]
        h0 = jnp.maximum(gd_r[:, :HC] + _dot(xj, a2_r[...]) + ge_r[...], 0.0)
        h1 = jnp.maximum(_dot(h0, w1_r[...]) + b1_r[...], 0.0)
        h2 = jnp.maximum(_dot(h1, w2_r[...]) + b2_r[...], 0.0)
        ap = ap_r[...]
        exs = []
        for h in range(H):
            sl = slice(h * C, (h + 1) * C)
            score = jnp.sum(h2[:, sl] * ap[:, sl], axis=1, keepdims=True)
            exh = jnp.exp(score)
            msg_r[:, sl] = xj[:, sl] * exh
            exs.append(exh)
        pad = jnp.zeros((BE, 16 - H), F32)
        ex16 = jnp.concatenate(exs + [pad], axis=1)
        ex_r[...] = ex16
        msg_r[:, HC:W] = jnp.concatenate(
            [ex16, jnp.zeros((BE, W - HC - 16), F32)], axis=1)
        e1 = jnp.maximum(_dot(h2, ew0_r[...]) + eb0_r[...], 0.0)
        e2 = jnp.maximum(_dot(e1, ew1_r[...]) + eb1_r[...], 0.0)
        e3 = jnp.maximum(_dot(e2, ew2_r[...]) + eb2_r[...], 0.0)
        eo_r[...] = el_r[...] + e3

    full = lambda s: pl.BlockSpec(s, lambda i: (0,) * len(s))
    wb = pl.BlockSpec((BE, W), lambda i: (i, 0))
    return pl.pallas_call(
        body,
        grid=(E // BE,),
        in_specs=[
            wb, wb,
            pl.BlockSpec((BE, HC), lambda i: (i, 0)),
            pl.BlockSpec((BE, C), lambda i: (i, 0)),
            full((W, HC)),
            full((HC, HC)), full((1, HC)), full((HC, HC)), full((1, HC)),
            full((1, HC)),
            full((HC, C)), full((1, C)), full((C, C)), full((1, C)),
            full((C, C)), full((1, C)),
        ],
        out_specs=[
            wb,
            pl.BlockSpec((BE, 16), lambda i: (i, 0)),
            pl.BlockSpec((BE, C), lambda i: (i, 0)),
        ],
        out_shape=[
            jax.ShapeDtypeStruct((E, W), F32),
            jax.ShapeDtypeStruct((E, 16), F32),
            jax.ShapeDtypeStruct((E, C), F32),
        ],
    )(gd, xj, ge, el, A02p, W1, b11, W2, b21, ap1,
      eW0, eb01, eW1, eb11, eW2, eb21)


def _tc_node_post(accP, degP, nlin, nW0, nb01, nW1, nb11, nW2, nb21, Wg):
    BN = 2000

    def body(acc_r, deg_r, nl_r, w0_r, b0_r, w1_r, b1_r, w2_r, b2_r,
             wg_r, y_r, xdd_r, dis_r, den_r):
        a = acc_r[0] + acc_r[1]
        cols = []
        for h in range(H):
            sl = slice(h * C, (h + 1) * C)
            cols.append(a[:, sl] / (a[:, HC + h:HC + h + 1] + 1e-16))
        agg = jnp.concatenate(cols, axis=1)
        n1 = jnp.maximum(_dot(agg, w0_r[...]) + b0_r[...], 0.0)
        n2 = jnp.maximum(_dot(n1, w1_r[...]) + b1_r[...], 0.0)
        n3 = jnp.maximum(_dot(n2, w2_r[...]) + b2_r[...], 0.0)
        node_out = n3 + nl_r[...]
        x = _dot(node_out, wg_r[...])
        deg = deg_r[0, :, 0:1] + deg_r[1, :, 0:1] + 1.0
        dis = deg ** -0.5
        y = dis * x
        y_r[...] = _pad128(y, BN)
        xdd_r[...] = dis * y
        dis_r[...] = dis
        den_r[...] = jnp.concatenate(
            [a[:, HC:HC + 16], jnp.zeros((BN, W - 16), F32)], axis=1)

    full = lambda s: pl.BlockSpec(s, lambda i: (0,) * len(s))
    return pl.pallas_call(
        body,
        grid=(N // BN,),
        in_specs=[
            pl.BlockSpec((NC, BN, W), lambda i: (0, i, 0)),
            pl.BlockSpec((NC, BN, W), lambda i: (0, i, 0)),
            pl.BlockSpec((BN, C), lambda i: (i, 0)),
            full((HC, C)), full((1, C)), full((C, C)), full((1, C)),
            full((C, C)), full((1, C)), full((C, C)),
        ],
        out_specs=[
            pl.BlockSpec((BN, W), lambda i: (i, 0)),
            pl.BlockSpec((BN, C), lambda i: (i, 0)),
            pl.BlockSpec((BN, 1), lambda i: (i, 0)),
            pl.BlockSpec((BN, W), lambda i: (i, 0)),
        ],
        out_shape=[
            jax.ShapeDtypeStruct((N, W), F32),
            jax.ShapeDtypeStruct((N, C), F32),
            jax.ShapeDtypeStruct((N, 1), F32),
            jax.ShapeDtypeStruct((N, W), F32),
        ],
    )(accP, degP, nlin, nW0, nb01, nW1, nb11, nW2, nb21, Wg)


def _tc_attn_w(ex16, dn, lw16):
    BE = 6400

    def body(ex_r, dn_r, lw_r, aw_r):
        ex = ex_r[...]
        acc = jnp.zeros((BE, 1), F32)
        for h in range(H):
            acc = acc + (ex[:, h:h + 1] / (dn_r[:, h:h + 1] + 1e-16)
                         * lw_r[0, h])
        aw_r[...] = acc

    return pl.pallas_call(
        body,
        grid=(E // BE,),
        in_specs=[
            pl.BlockSpec((BE, 16), lambda i: (i, 0)),
            pl.BlockSpec((BE, W), lambda i: (i, 0)),
            pl.BlockSpec((1, 16), lambda i: (0, 0)),
        ],
        out_specs=pl.BlockSpec((BE, 1), lambda i: (i, 0)),
        out_shape=jax.ShapeDtypeStruct((E, 1), F32),
    )(ex16, dn, lw16)


def _tc_final(zP, dis, xdd, bg1):
    BN = 2000

    def body(z_r, dis_r, xdd_r, bg_r, o_r):
        z = z_r[0, :, :C] + z_r[1, :, :C]
        o_r[...] = dis_r[...] * z + xdd_r[...] + bg_r[...]

    return pl.pallas_call(
        body,
        grid=(N // BN,),
        in_specs=[
            pl.BlockSpec((NC, BN, W), lambda i: (0, i, 0)),
            pl.BlockSpec((BN, 1), lambda i: (i, 0)),
            pl.BlockSpec((BN, C), lambda i: (i, 0)),
            pl.BlockSpec((1, C), lambda i: (0, 0)),
        ],
        out_specs=pl.BlockSpec((BN, C), lambda i: (i, 0)),
        out_shape=jax.ShapeDtypeStruct((N, C), F32),
    )(zP, dis, xdd, bg1)


def kernel(node_features, edge_index, edge_features, Wn, bn, We, be, Wi, Wj,
           Weij, attn_W0, attn_b0, attn_W1, attn_b1, attn_W2, attn_b2,
           attn_param, node_W0, node_b0, node_W1, node_b1, node_W2, node_b2,
           edge_W0, edge_b0, edge_W1, edge_b1, edge_W2, edge_b2, lin_W,
           Wg, bg):
    src2 = edge_index[0].reshape(IR, GW)
    dst2 = edge_index[1].reshape(IR, GW)

    A00 = attn_W0[:HC]
    A01 = attn_W0[HC:2 * HC]
    A02p = jnp.concatenate(
        [attn_W0[2 * HC:], jnp.zeros((W - HC, HC), F32)], axis=0)
    b01 = attn_b0.reshape(1, HC)
    b11 = attn_b1.reshape(1, HC)
    b21 = attn_b2.reshape(1, HC)
    ap1 = attn_param.reshape(1, HC)
    bn1 = bn.reshape(1, C)
    be1 = be.reshape(1, C)
    bg1 = bg.reshape(1, C)
    lw16 = jnp.zeros((1, 16), F32).at[0, :H].set(lin_W[:, 0])
    zrows = jnp.zeros((ACC // NS, W), F32)
    ones_h = jnp.zeros((GW, W), F32).at[:, 0].set(1.0)

    td, tj, nlin = _tc_node_pre(node_features, Wj, A00, Wi, Wn, bn1)
    ge, el = _tc_edge_pre(edge_features, Weij, A01, We, b01, be1)
    srcp2, dstp2 = _tc_redirect(src2, dst2)
    gd, xj = _sc_gather2(td, tj, dst2, src2)
    degP = _sc_degree(srcp2, zrows, ones_h)
    msg, ex16, eo = _tc_edge_mlp(
        gd, xj, ge, el, A02p, attn_W1, b11, attn_W2, b21, ap1,
        edge_W0, edge_b0.reshape(1, C), edge_W1, edge_b1.reshape(1, C),
        edge_W2, edge_b2.reshape(1, C))
    accP = _sc_scatter_rows(msg, dst2, zrows)
    y, xdd, dis, denT = _tc_node_post(
        accP, degP, nlin, node_W0, node_b0.reshape(1, C),
        node_W1, node_b1.reshape(1, C), node_W2, node_b2.reshape(1, C), Wg)
    dn = _sc_gather1(denT, dst2)
    aw = _tc_attn_w(ex16, dn, lw16)
    zP = _sc_spmm(y, src2, dstp2, zrows)
    out = _tc_final(zP, dis, xdd, bg1)
    return (out, eo, aw.reshape(E))


# speedup vs baseline: 15.4782x; 1.0137x over previous
import functools

import jax
import jax.numpy as jnp
from jax import lax
from jax.experimental import pallas as pl
from jax.experimental.pallas import tpu as pltpu
from jax.experimental.pallas import tpu_sc as plsc

N = 10000
E = 320000
DN = 128
DE = 16
C = 32
H = 3
HC = H * C
W = 128

NC, NS = 2, 16
GW = 128
IR = E // GW
ACC = 10240
TRASH = N

PREC = lax.Precision.HIGHEST
F32 = jnp.float32


def _mesh():
    return plsc.VectorSubcoreMesh(
        core_axis_name="c", subcore_axis_name="s", num_cores=NC, num_subcores=NS
    )


def _dot(a, b):
    return jnp.dot(a, b, preferred_element_type=F32, precision=PREC)


def _pad128(x, bn):
    return jnp.concatenate([x, jnp.zeros((bn, W - x.shape[1]), F32)], axis=1)


def _sc_gather2(td, tj, dst2, src2):
    @functools.partial(
        pl.kernel,
        out_type=(jax.ShapeDtypeStruct((E, W), F32),) * 2,
        mesh=_mesh(),
    )
    def k(td_h, tj_h, d_h, s_h, gd_h, xj_h):
        def body(d_v, s_v, gd_v, xj_v):
            pltpu.sync_copy(td_h.at[d_v.at[0]], gd_v)
            pltpu.sync_copy(tj_h.at[s_v.at[0]], xj_v)

        pltpu.emit_pipeline(
            body,
            grid=(IR,),
            in_specs=[
                pl.BlockSpec((1, GW), lambda i: (i, 0)),
                pl.BlockSpec((1, GW), lambda i: (i, 0)),
            ],
            out_specs=[pl.BlockSpec((GW, W), lambda i: (i, 0))] * 2,
            core_axis_name=("c", "s"),
            dimension_semantics=(pltpu.PARALLEL,),
        )(d_h, s_h, gd_h, xj_h)

    return k(td, tj, dst2, src2)


def _sc_gather1(tab, idx2):
    @functools.partial(
        pl.kernel,
        out_type=jax.ShapeDtypeStruct((E, W), F32),
        mesh=_mesh(),
    )
    def k(t_h, i_h, o_h):
        def body(i_v, o_v):
            pltpu.sync_copy(t_h.at[i_v.at[0]], o_v)

        pltpu.emit_pipeline(
            body,
            grid=(IR,),
            in_specs=[pl.BlockSpec((1, GW), lambda i: (i, 0))],
            out_specs=[pl.BlockSpec((GW, W), lambda i: (i, 0))],
            core_axis_name=("c", "s"),
            dimension_semantics=(pltpu.PARALLEL,),
        )(i_h, o_h)

    return k(tab, idx2)


def _sc_scatter_rows(msg, dst2, zrows):
    @functools.partial(
        pl.kernel,
        out_type=jax.ShapeDtypeStruct((NC, ACC, W), F32),
        mesh=_mesh(),
        scratch_types=[pltpu.VMEM_SHARED((ACC, W), F32)],
    )
    def k(msg_h, d_h, z_h, acc_o, acc_s):
        cid = lax.axis_index("c")
        sid = lax.axis_index("s")
        r0 = sid * (ACC // NS)
        pltpu.sync_copy(z_h, acc_s.at[pl.ds(r0, ACC // NS)])
        plsc.subcore_barrier()

        def body(msg_v, d_v):
            pltpu.sync_copy(msg_v, acc_s.at[d_v.at[0]], add=True)

        pltpu.emit_pipeline(
            body,
            grid=(IR,),
            in_specs=[
                pl.BlockSpec((GW, W), lambda i: (i, 0)),
                pl.BlockSpec((1, GW), lambda i: (i, 0)),
            ],
            core_axis_name=("c", "s"),
            dimension_semantics=(pltpu.PARALLEL,),
        )(msg_h, d_h)
        plsc.subcore_barrier()
        pltpu.sync_copy(acc_s.at[pl.ds(r0, ACC // NS)],
                        acc_o.at[cid, pl.ds(r0, ACC // NS)])

    return k(msg, dst2, zrows)


def _sc_degree(srcp2, zrows, ones_h):
    @functools.partial(
        pl.kernel,
        out_type=jax.ShapeDtypeStruct((NC, ACC, W), F32),
        mesh=_mesh(),
        scratch_types=[
            pltpu.VMEM_SHARED((ACC, W), F32),
            pltpu.VMEM((GW, W), F32),
        ],
    )
    def k(sp_h, z_h, o_h, deg_o, deg_s, ones_v):
        cid = lax.axis_index("c")
        sid = lax.axis_index("s")
        r0 = sid * (ACC // NS)
        pltpu.sync_copy(z_h, deg_s.at[pl.ds(r0, ACC // NS)])
        pltpu.sync_copy(o_h, ones_v)
        plsc.subcore_barrier()

        def body(sp_v):
            pltpu.sync_copy(ones_v, deg_s.at[sp_v.at[0]], add=True)

        pltpu.emit_pipeline(
            body,
            grid=(IR,),
            in_specs=[pl.BlockSpec((1, GW), lambda i: (i, 0))],
            core_axis_name=("c", "s"),
            dimension_semantics=(pltpu.PARALLEL,),
        )(sp_h)
        plsc.subcore_barrier()
        pltpu.sync_copy(deg_s.at[pl.ds(r0, ACC // NS)],
                        deg_o.at[cid, pl.ds(r0, ACC // NS)])

    return k(srcp2, zrows, ones_h)


def _sc_spmm(y, src2, dstp2, zrows):
    @functools.partial(
        pl.kernel,
        out_type=jax.ShapeDtypeStruct((NC, ACC, W), F32),
        mesh=_mesh(),
        scratch_types=[
            pltpu.VMEM_SHARED((ACC, W), F32),
            pltpu.VMEM((GW, W), F32),
        ],
    )
    def k(y_h, s_h, d_h, z_h, z_o, z_s, buf_v):
        cid = lax.axis_index("c")
        sid = lax.axis_index("s")
        r0 = sid * (ACC // NS)
        pltpu.sync_copy(z_h, z_s.at[pl.ds(r0, ACC // NS)])
        plsc.subcore_barrier()

        def body(s_v, d_v):
            pltpu.sync_copy(y_h.at[s_v.at[0]], buf_v)
            pltpu.sync_copy(buf_v, z_s.at[d_v.at[0]], add=True)

        pltpu.emit_pipeline(
            body,
            grid=(IR,),
            in_specs=[
                pl.BlockSpec((1, GW), lambda i: (i, 0)),
                pl.BlockSpec((1, GW), lambda i: (i, 0)),
            ],
            core_axis_name=("c", "s"),
            dimension_semantics=(pltpu.PARALLEL,),
        )(s_h, d_h)
        plsc.subcore_barrier()
        pltpu.sync_copy(z_s.at[pl.ds(r0, ACC // NS)],
                        z_o.at[cid, pl.ds(r0, ACC // NS)])

    return k(y, src2, dstp2, zrows)


def _tc_node_pre(nf, Wd, Wi, Wn, bn1):
    BN = 2000

    def body(nf_r, wd_r, wi_r, wn_r, bn_r, td_r, tj_r, nl_r):
        x = nf_r[...]
        td_r[...] = _pad128(_dot(x, wd_r[...]), BN)
        tj_r[...] = _pad128(_dot(x, wi_r[...]), BN)
        nl_r[...] = _dot(x, wn_r[...]) + bn_r[...]

    full = lambda s: pl.BlockSpec(s, lambda i: (0,) * len(s))
    return pl.pallas_call(
        body,
        grid=(N // BN,),
        in_specs=[
            pl.BlockSpec((BN, DN), lambda i: (i, 0)),
            full((DN, HC)), full((DN, HC)),
            full((DN, C)), full((1, C)),
        ],
        out_specs=[
            pl.BlockSpec((BN, W), lambda i: (i, 0)),
            pl.BlockSpec((BN, W), lambda i: (i, 0)),
            pl.BlockSpec((BN, C), lambda i: (i, 0)),
        ],
        out_shape=[
            jax.ShapeDtypeStruct((N, W), F32),
            jax.ShapeDtypeStruct((N, W), F32),
            jax.ShapeDtypeStruct((N, C), F32),
        ],
    )(nf, Wd, Wi, Wn, bn1)


def _tc_edge_pre(ef, We0, We, b01, be1):
    BE = 6400

    def body(ef_r, we0_r, we_r, b0_r, be_r, ge_r, el_r):
        x = ef_r[...]
        ge_r[...] = _dot(x, we0_r[...]) + b0_r[...]
        el_r[...] = _dot(x, we_r[...]) + be_r[...]

    full = lambda s: pl.BlockSpec(s, lambda i: (0,) * len(s))
    return pl.pallas_call(
        body,
        grid=(E // BE,),
        in_specs=[
            pl.BlockSpec((BE, DE), lambda i: (i, 0)),
            full((DE, HC)), full((DE, C)),
            full((1, HC)), full((1, C)),
        ],
        out_specs=[
            pl.BlockSpec((BE, HC), lambda i: (i, 0)),
            pl.BlockSpec((BE, C), lambda i: (i, 0)),
        ],
        out_shape=[
            jax.ShapeDtypeStruct((E, HC), F32),
            jax.ShapeDtypeStruct((E, C), F32),
        ],
    )(ef, We0, We, b01, be1)


def _tc_redirect(src2, dst2):
    def body(s_r, d_r, sp_r, dp_r):
        s = s_r[...]
        d = d_r[...]
        loop = s == d
        sp_r[...] = jnp.where(loop, TRASH, s)
        dp_r[...] = jnp.where(loop, TRASH, d)

    whole = pl.BlockSpec((IR, GW), lambda: (0, 0))
    return pl.pallas_call(
        body,
        grid=(),
        in_specs=[whole, whole],
        out_specs=[whole, whole],
        out_shape=[jax.ShapeDtypeStruct((IR, GW), jnp.int32)] * 2,
    )(src2, dst2)


def _tc_edge_mlp(gd, xj, ge, el, A02p, W1, b11, W2, b21, ap1,
                 eW0, eb01, eW1, eb11, eW2, eb21):
    BE = 1600

    def body(gd_r, xj_r, ge_r, el_r, a2_r, w1_r, b1_r, w2_r, b2_r, ap_r,
             ew0_r, eb0_r, ew1_r, eb1_r, ew2_r, eb2_r,
             msg_r, ex_r, eo_r):
        xj = xj_r[...]
        h0 = jnp.maximum(gd_r[:, :HC] + _dot(xj, a2_r[...]) + ge_r[...], 0.0)
        h1 = jnp.maximum(_dot(h0, w1_r[...]) + b1_r[...], 0.0)
        h2 = jnp.maximum(_dot(h1, w2_r[...]) + b2_r[...], 0.0)
        ap = ap_r[...]
        exs = []
        for h in range(H):
            sl = slice(h * C, (h + 1) * C)
            score = jnp.sum(h2[:, sl] * ap[:, sl], axis=1, keepdims=True)
            exh = jnp.exp(score)
            msg_r[:, sl] = xj[:, sl] * exh
            exs.append(exh)
        pad = jnp.zeros((BE, 16 - H), F32)
        ex16 = jnp.concatenate(exs + [pad], axis=1)
        ex_r[...] = ex16
        msg_r[:, HC:W] = jnp.concatenate(
            [ex16, jnp.zeros((BE, W - HC - 16), F32)], axis=1)
        e1 = jnp.maximum(_dot(h2, ew0_r[...]) + eb0_r[...], 0.0)
        e2 = jnp.maximum(_dot(e1, ew1_r[...]) + eb1_r[...], 0.0)
        e3 = jnp.maximum(_dot(e2, ew2_r[...]) + eb2_r[...], 0.0)
        eo_r[...] = el_r[...] + e3

    full = lambda s: pl.BlockSpec(s, lambda i: (0,) * len(s))
    wb = pl.BlockSpec((BE, W), lambda i: (i, 0))
    return pl.pallas_call(
        body,
        grid=(E // BE,),
        in_specs=[
            wb, wb,
            pl.BlockSpec((BE, HC), lambda i: (i, 0)),
            pl.BlockSpec((BE, C), lambda i: (i, 0)),
            full((W, HC)),
            full((HC, HC)), full((1, HC)), full((HC, HC)), full((1, HC)),
            full((1, HC)),
            full((HC, C)), full((1, C)), full((C, C)), full((1, C)),
            full((C, C)), full((1, C)),
        ],
        out_specs=[
            wb,
            pl.BlockSpec((BE, 16), lambda i: (i, 0)),
            pl.BlockSpec((BE, C), lambda i: (i, 0)),
        ],
        out_shape=[
            jax.ShapeDtypeStruct((E, W), F32),
            jax.ShapeDtypeStruct((E, 16), F32),
            jax.ShapeDtypeStruct((E, C), F32),
        ],
    )(gd, xj, ge, el, A02p, W1, b11, W2, b21, ap1,
      eW0, eb01, eW1, eb11, eW2, eb21)


def _tc_node_post(accP, degP, nlin, nW0, nb01, nW1, nb11, nW2, nb21, Wg):
    BN = 2000

    def body(acc_r, deg_r, nl_r, w0_r, b0_r, w1_r, b1_r, w2_r, b2_r,
             wg_r, y_r, xdd_r, dis_r, den_r):
        a = acc_r[0] + acc_r[1]
        cols = []
        for h in range(H):
            sl = slice(h * C, (h + 1) * C)
            cols.append(a[:, sl] / (a[:, HC + h:HC + h + 1] + 1e-16))
        agg = jnp.concatenate(cols, axis=1)
        n1 = jnp.maximum(_dot(agg, w0_r[...]) + b0_r[...], 0.0)
        n2 = jnp.maximum(_dot(n1, w1_r[...]) + b1_r[...], 0.0)
        n3 = jnp.maximum(_dot(n2, w2_r[...]) + b2_r[...], 0.0)
        node_out = n3 + nl_r[...]
        x = _dot(node_out, wg_r[...])
        deg = deg_r[0, :, 0:1] + deg_r[1, :, 0:1] + 1.0
        dis = deg ** -0.5
        y = dis * x
        y_r[...] = _pad128(y, BN)
        xdd_r[...] = dis * y
        dis_r[...] = dis
        den_r[...] = jnp.concatenate(
            [a[:, HC:HC + 16], jnp.zeros((BN, W - 16), F32)], axis=1)

    full = lambda s: pl.BlockSpec(s, lambda i: (0,) * len(s))
    return pl.pallas_call(
        body,
        grid=(N // BN,),
        in_specs=[
            pl.BlockSpec((NC, BN, W), lambda i: (0, i, 0)),
            pl.BlockSpec((NC, BN, W), lambda i: (0, i, 0)),
            pl.BlockSpec((BN, C), lambda i: (i, 0)),
            full((HC, C)), full((1, C)), full((C, C)), full((1, C)),
            full((C, C)), full((1, C)), full((C, C)),
        ],
        out_specs=[
            pl.BlockSpec((BN, W), lambda i: (i, 0)),
            pl.BlockSpec((BN, C), lambda i: (i, 0)),
            pl.BlockSpec((BN, 1), lambda i: (i, 0)),
            pl.BlockSpec((BN, W), lambda i: (i, 0)),
        ],
        out_shape=[
            jax.ShapeDtypeStruct((N, W), F32),
            jax.ShapeDtypeStruct((N, C), F32),
            jax.ShapeDtypeStruct((N, 1), F32),
            jax.ShapeDtypeStruct((N, W), F32),
        ],
    )(accP, degP, nlin, nW0, nb01, nW1, nb11, nW2, nb21, Wg)


def _tc_attn_w(ex16, dn, lw16):
    BE = 6400

    def body(ex_r, dn_r, lw_r, aw_r):
        ex = ex_r[...]
        acc = jnp.zeros((BE, 1), F32)
        for h in range(H):
            acc = acc + (ex[:, h:h + 1] / (dn_r[:, h:h + 1] + 1e-16)
                         * lw_r[0, h])
        aw_r[...] = acc

    return pl.pallas_call(
        body,
        grid=(E // BE,),
        in_specs=[
            pl.BlockSpec((BE, 16), lambda i: (i, 0)),
            pl.BlockSpec((BE, W), lambda i: (i, 0)),
            pl.BlockSpec((1, 16), lambda i: (0, 0)),
        ],
        out_specs=pl.BlockSpec((BE, 1), lambda i: (i, 0)),
        out_shape=jax.ShapeDtypeStruct((E, 1), F32),
    )(ex16, dn, lw16)


def _tc_final(zP, dis, xdd, bg1):
    BN = 2000

    def body(z_r, dis_r, xdd_r, bg_r, o_r):
        z = z_r[0, :, :C] + z_r[1, :, :C]
        o_r[...] = dis_r[...] * z + xdd_r[...] + bg_r[...]

    return pl.pallas_call(
        body,
        grid=(N // BN,),
        in_specs=[
            pl.BlockSpec((NC, BN, W), lambda i: (0, i, 0)),
            pl.BlockSpec((BN, 1), lambda i: (i, 0)),
            pl.BlockSpec((BN, C), lambda i: (i, 0)),
            pl.BlockSpec((1, C), lambda i: (0, 0)),
        ],
        out_specs=pl.BlockSpec((BN, C), lambda i: (i, 0)),
        out_shape=jax.ShapeDtypeStruct((N, C), F32),
    )(zP, dis, xdd, bg1)


def kernel(node_features, edge_index, edge_features, Wn, bn, We, be, Wi, Wj,
           Weij, attn_W0, attn_b0, attn_W1, attn_b1, attn_W2, attn_b2,
           attn_param, node_W0, node_b0, node_W1, node_b1, node_W2, node_b2,
           edge_W0, edge_b0, edge_W1, edge_b1, edge_W2, edge_b2, lin_W,
           Wg, bg):
    src2 = edge_index[0].reshape(IR, GW)
    dst2 = edge_index[1].reshape(IR, GW)

    Wd = jnp.dot(Wj, attn_W0[:HC], preferred_element_type=F32,
                 precision=lax.Precision.HIGHEST)
    We0 = jnp.dot(Weij, attn_W0[HC:2 * HC], preferred_element_type=F32,
                  precision=lax.Precision.HIGHEST)
    A02p = jnp.concatenate(
        [attn_W0[2 * HC:], jnp.zeros((W - HC, HC), F32)], axis=0)
    b01 = attn_b0.reshape(1, HC)
    b11 = attn_b1.reshape(1, HC)
    b21 = attn_b2.reshape(1, HC)
    ap1 = attn_param.reshape(1, HC)
    bn1 = bn.reshape(1, C)
    be1 = be.reshape(1, C)
    bg1 = bg.reshape(1, C)
    lw16 = jnp.zeros((1, 16), F32).at[0, :H].set(lin_W[:, 0])
    zrows = jnp.zeros((ACC // NS, W), F32)
    ones_h = jnp.zeros((GW, W), F32).at[:, 0].set(1.0)

    td, tj, nlin = _tc_node_pre(node_features, Wd, Wi, Wn, bn1)
    ge, el = _tc_edge_pre(edge_features, We0, We, b01, be1)
    srcp2, dstp2 = _tc_redirect(src2, dst2)
    gd, xj = _sc_gather2(td, tj, dst2, src2)
    degP = _sc_degree(srcp2, zrows, ones_h)
    msg, ex16, eo = _tc_edge_mlp(
        gd, xj, ge, el, A02p, attn_W1, b11, attn_W2, b21, ap1,
        edge_W0, edge_b0.reshape(1, C), edge_W1, edge_b1.reshape(1, C),
        edge_W2, edge_b2.reshape(1, C))
    accP = _sc_scatter_rows(msg, dst2, zrows)
    y, xdd, dis, denT = _tc_node_post(
        accP, degP, nlin, node_W0, node_b0.reshape(1, C),
        node_W1, node_b1.reshape(1, C), node_W2, node_b2.reshape(1, C), Wg)
    dn = _sc_gather1(denT, dst2)
    aw = _tc_attn_w(ex16, dn, lw16)
    zP = _sc_spmm(y, src2, dstp2, zrows)
    out = _tc_final(zP, dis, xdd, bg1)
    return (out, eo, aw.reshape(E))
